# Initial kernel scaffold; baseline (speedup 1.0000x reference)
import jax
import jax.numpy as jnp
from jax.experimental import pallas as pl


def kernel(x, pe, params, edge_index, batch):
    raise NotImplementedError("write your pallas kernel here")


# speedup vs baseline: 4.2726x; 4.2726x over previous
import functools

import jax
import jax.numpy as jnp
import numpy as np
from jax import lax
from jax.experimental import pallas as pl
from jax.experimental.pallas import tpu as pltpu
from jax.experimental.pallas import tpu_sc as plsc

N = 10000
E = 320000
H = 64
HEADS = 4
DH = H // HEADS
G = 16
OUT_DIM = 128
NUM_LAYERS = 5

NC = 2
NS = 16
NW = NC * NS
CH = 128
NCH = 80
EP = NW * NCH * CH
ND = 10112
ROWS_PER_SUB = ND // NS

BT = 2000
BTE = 4096
NPAD = 10240
TQ = 1024
TK = 1024
NT = NPAD // TQ


def _dot(a, b):
    return jnp.dot(a, b, preferred_element_type=jnp.float32)


def _enc_body(inp_ref, x2_ref, ew1, eb1, ew2, eb2, wa, ba, wb,
              h_ref, ab_ref, tf_ref):
    inp = inp_ref[...]
    t = jnp.clip(jnp.round(x2_ref[...] * 3.0), 1.0, 3.0) - 1.0
    h = jnp.zeros_like(h_ref)
    for i in range(3):
        hid = jnp.maximum(_dot(inp, ew1[i]) + eb1[i], 0.0)
        cand = _dot(hid, ew2[i]) + eb2[i]
        h = jnp.where(t == jnp.float32(i), cand, h)
    h_ref[...] = h
    ab_ref[...] = jnp.concatenate(
        [_dot(h, wa[...]) + ba[...], _dot(h, wb[...])], axis=1)
    tf_ref[...] = t


def _encode(inp, x2, ew1, eb1, ew2, eb2, wa, ba, wb):
    grid = (N // BT,)
    row = lambda i: (i, 0)
    const = lambda shape: pl.BlockSpec(shape, lambda i: (0,) * len(shape))
    return pl.pallas_call(
        _enc_body,
        grid=grid,
        in_specs=[
            pl.BlockSpec((BT, inp.shape[1]), row),
            pl.BlockSpec((BT, 1), row),
            const((3, inp.shape[1], H)), const((3, 1, H)),
            const((3, H, H)), const((3, 1, H)),
            const((H, H)), const((1, H)), const((H, H)),
        ],
        out_specs=[
            pl.BlockSpec((BT, H), row), pl.BlockSpec((BT, 2 * H), row),
            pl.BlockSpec((BT, 1), row),
        ],
        out_shape=[
            jax.ShapeDtypeStruct((N, H), jnp.float32),
            jax.ShapeDtypeStruct((N, 2 * H), jnp.float32),
            jax.ShapeDtypeStruct((N, 1), jnp.float32),
        ],
    )(inp, x2, ew1, eb1, ew2, eb2, wa, ba, wb)


def _edge_first_body(a_ref, b_ref, w1c, w2, b2, eb, o_ref):
    c0 = _dot(eb[...], w1c[...])
    hid = jnp.maximum(a_ref[:, :H] + b_ref[:, H:] + c0, 0.0)
    o_ref[...] = eb[...] + _dot(hid, w2[...]) + b2[...]


def _edge_body(a_ref, b_ref, he_ref, w1c, w2, b2, o_ref):
    he = he_ref[...]
    hid = jnp.maximum(a_ref[:, :H] + b_ref[:, H:] + _dot(he, w1c[...]), 0.0)
    o_ref[...] = he + _dot(hid, w2[...]) + b2[...]


def _edge_update(gsrc, gdst, he, w1c, w2, b2, eb):
    grid = (EP // BTE,)
    row = lambda i: (i, 0)
    const = lambda shape: pl.BlockSpec(shape, lambda i: (0,) * len(shape))
    tile = pl.BlockSpec((BTE, H), row)
    a_spec = pl.BlockSpec((BTE, 2 * H), row)
    b_spec = pl.BlockSpec((BTE, 2 * H), row)
    if he is None:
        return pl.pallas_call(
            _edge_first_body, grid=grid,
            in_specs=[a_spec, b_spec, const((H, H)), const((H, H)),
                      const((1, H)), const((1, H))],
            out_specs=tile,
            out_shape=jax.ShapeDtypeStruct((EP, H), jnp.float32),
        )(gsrc, gdst, w1c, w2, b2, eb)
    return pl.pallas_call(
        _edge_body, grid=grid,
        in_specs=[a_spec, b_spec, tile, const((H, H)), const((H, H)),
                  const((1, H))],
        out_specs=tile,
        out_shape=jax.ShapeDtypeStruct((EP, H), jnp.float32),
    )(gsrc, gdst, he, w1c, w2, b2)


def _node_body(h_ref, p0_ref, p1_ref, v1a, v1b, nb1, v2, nb2, wa, ba, wb,
               h_out, ab_out):
    h = h_ref[...]
    m = p0_ref[...] + p1_ref[...]
    hid = jnp.maximum(_dot(h, v1a[...]) + _dot(m, v1b[...]) + nb1[...], 0.0)
    hn = h + _dot(hid, v2[...]) + nb2[...]
    h_out[...] = hn
    ab_out[...] = jnp.concatenate(
        [_dot(hn, wa[...]) + ba[...], _dot(hn, wb[...])], axis=1)


def _node_last_body(h_ref, p0_ref, p1_ref, v1a, v1b, nb1, v2, nb2, win, bin_,
                    h_out, qkv_out):
    h = h_ref[...]
    m = p0_ref[...] + p1_ref[...]
    hid = jnp.maximum(_dot(h, v1a[...]) + _dot(m, v1b[...]) + nb1[...], 0.0)
    hn = h + _dot(hid, v2[...]) + nb2[...]
    h_out[...] = hn
    qkv_out[...] = _dot(hn, win[...]) + bin_[...]


def _node_update(h, p0, p1, v1a, v1b, nb1, v2, nb2, last, *proj):
    grid = (N // BT,)
    row = lambda i: (i, 0)
    const = lambda shape: pl.BlockSpec(shape, lambda i: (0,) * len(shape))
    tile = pl.BlockSpec((BT, H), row)
    if not last:
        wa, ba, wb = proj
        return pl.pallas_call(
            _node_body, grid=grid,
            in_specs=[tile, tile, tile, const((H, H)), const((H, H)),
                      const((1, H)), const((H, H)), const((1, H)),
                      const((H, H)), const((1, H)), const((H, H))],
            out_specs=[tile, pl.BlockSpec((BT, 2 * H), row)],
            out_shape=[jax.ShapeDtypeStruct((N, H), jnp.float32),
                       jax.ShapeDtypeStruct((N, 2 * H), jnp.float32)],
        )(h, p0, p1, v1a, v1b, nb1, v2, nb2, wa, ba, wb)
    win, bin_ = proj
    return pl.pallas_call(
        _node_last_body, grid=grid,
        in_specs=[tile, tile, tile, const((H, H)), const((H, H)),
                  const((1, H)), const((H, H)), const((1, H)),
                  const((H, 3 * H)), const((1, 3 * H))],
        out_specs=[tile, pl.BlockSpec((BT, 3 * H), row)],
        out_shape=[jax.ShapeDtypeStruct((N, H), jnp.float32),
                   jax.ShapeDtypeStruct((N, 3 * H), jnp.float32)],
    )(h, p0, p1, v1a, v1b, nb1, v2, nb2, win, bin_)


def _attn_body(lo_ref, hi_ref, qkv_ref, h_ref, tf_ref, bq_ref, br_ref,
               wout, bout, wf1, bf1, wf2, bf2, wd1, bd1, wd2, bd2, o_ref):
    t = pl.program_id(0)
    lo = lo_ref[t]
    hi = hi_ref[t]
    bq = bq_ref[...]
    scale = jnp.float32(1.0 / np.sqrt(DH))
    qt = qkv_ref[pl.ds(pl.multiple_of(t * TQ, TQ), TQ), :]
    qs = [qt[:, hd * DH:(hd + 1) * DH] * scale for hd in range(HEADS)]

    def body(j, carry):
        off = pl.multiple_of(j * TK, TK)
        blk = qkv_ref[pl.ds(off, TK), :]
        bk = br_ref[:, pl.ds(off, TK)]
        mask = bq == bk
        new = []
        for hd in range(HEADS):
            m, l_, acc = carry[hd]
            kh = blk[:, H + hd * DH:H + (hd + 1) * DH]
            vh = blk[:, 2 * H + hd * DH:2 * H + (hd + 1) * DH]
            s = lax.dot_general(qs[hd], kh, (((1,), (1,)), ((), ())),
                                preferred_element_type=jnp.float32)
            s = jnp.where(mask, s, jnp.float32(-1e9))
            mn = jnp.maximum(m, jnp.max(s, axis=1, keepdims=True))
            alpha = jnp.exp(m - mn)
            p = jnp.exp(s - mn)
            l2 = l_ * alpha + jnp.sum(p, axis=1, keepdims=True)
            acc2 = acc * alpha + _dot(p, vh)
            new.append((mn, l2, acc2))
        return tuple(new)

    init = tuple((jnp.full((TQ, 1), -1e30, jnp.float32),
                  jnp.zeros((TQ, 1), jnp.float32),
                  jnp.zeros((TQ, DH), jnp.float32)) for _ in range(HEADS))
    carry = lax.fori_loop(lo, hi, body, init)
    o = jnp.concatenate([acc / l_ for (m, l_, acc) in carry], axis=1)
    h = h_ref[...]
    ao = _dot(o, wout[...]) + bout[...]
    z = h + h + ao
    hf = _dot(jnp.maximum(_dot(z, wf1[...]) + bf1[...], 0.0), wf2[...]) + bf2[...]
    tval = tf_ref[...]
    res = jnp.zeros_like(o_ref)
    for i in range(3):
        cand = _dot(jnp.maximum(_dot(hf, wd1[i]) + bd1[i], 0.0), wd2[i]) + bd2[i]
        res = jnp.where(tval == jnp.float32(i), cand, res)
    o_ref[...] = res


def _attention(lo, hi, qkv, h, tf, bq, br, wout, bout, wf1, bf1, wf2, bf2,
               wd1, bd1, wd2, bd2):
    row = lambda i, *_: (i, 0)
    const = lambda shape: pl.BlockSpec(shape, lambda i, *_: (0,) * len(shape))
    grid_spec = pltpu.PrefetchScalarGridSpec(
        num_scalar_prefetch=2,
        grid=(NT,),
        in_specs=[
            const((NPAD, 3 * H)),
            pl.BlockSpec((TQ, H), row),
            pl.BlockSpec((TQ, 1), row),
            pl.BlockSpec((TQ, 1), row),
            const((1, NPAD)),
            const((H, H)), const((1, H)),
            const((H, H)), const((1, H)),
            const((H, H)), const((1, H)),
            const((3, H, H)), const((3, 1, H)),
            const((3, H, OUT_DIM)), const((3, 1, OUT_DIM)),
        ],
        out_specs=pl.BlockSpec((TQ, OUT_DIM), row),
    )
    return pl.pallas_call(
        _attn_body,
        grid_spec=grid_spec,
        out_shape=jax.ShapeDtypeStruct((NPAD, OUT_DIM), jnp.float32),
    )(lo, hi, qkv, h, tf, bq, br, wout, bout, wf1, bf1, wf2, bf2,
      wd1, bd1, wd2, bd2)


@functools.lru_cache(maxsize=1)
def _sc_gather2_kernel():
    mesh = plsc.VectorSubcoreMesh(core_axis_name="c", subcore_axis_name="s")

    @functools.partial(
        pl.kernel,
        mesh=mesh,
        out_type=[jax.ShapeDtypeStruct((EP, 2 * H), jnp.float32),
                  jax.ShapeDtypeStruct((EP, 2 * H), jnp.float32)],
        scratch_types=[
            pltpu.VMEM((NCH, CH), jnp.int32),
            pltpu.VMEM((NCH, CH), jnp.int32),
            pltpu.VMEM((CH, 2 * H), jnp.float32),
            pltpu.VMEM((CH, 2 * H), jnp.float32),
            pltpu.SemaphoreType.DMA,
            pltpu.SemaphoreType.DMA,
        ],
    )
    def k(tab, isrc, idst, oa, ob, iv_s, iv_d, bufa, bufb, sema, semb):
        wid = lax.axis_index("s") * NC + lax.axis_index("c")
        base = wid * (NCH * CH)
        pltpu.sync_copy(isrc.at[wid], iv_s)
        pltpu.sync_copy(idst.at[wid], iv_d)

        def body(j, carry):
            ca = pltpu.async_copy(tab.at[iv_s.at[j]], bufa, sema)
            cb = pltpu.async_copy(tab.at[iv_d.at[j]], bufb, semb)
            ca.wait()
            pltpu.sync_copy(bufa, oa.at[pl.ds(base + j * CH, CH)])
            cb.wait()
            pltpu.sync_copy(bufb, ob.at[pl.ds(base + j * CH, CH)])
            return carry

        lax.fori_loop(0, NCH, body, 0)

    return k


def _sc_gather2(tab, isrc, idst):
    return _sc_gather2_kernel()(tab, isrc, idst)


@functools.lru_cache(maxsize=1)
def _sc_scatter_kernel():
    mesh = plsc.VectorSubcoreMesh(core_axis_name="c", subcore_axis_name="s")

    @functools.partial(
        pl.kernel,
        mesh=mesh,
        out_type=jax.ShapeDtypeStruct((NC, ND, H), jnp.float32),
        scratch_types=[
            pltpu.VMEM((NCH, CH), jnp.int32),
            pltpu.VMEM((CH, H), jnp.float32),
            pltpu.VMEM_SHARED((ND, H), jnp.float32),
        ],
    )
    def k(he, idst, zeros_nd, out, iv, buf, acc):
        cid = lax.axis_index("c")
        sid = lax.axis_index("s")
        wid = sid * NC + cid
        pltpu.sync_copy(zeros_nd.at[pl.ds(sid * ROWS_PER_SUB, ROWS_PER_SUB)],
                        acc.at[pl.ds(sid * ROWS_PER_SUB, ROWS_PER_SUB)])
        pltpu.sync_copy(idst.at[wid], iv)
        plsc.subcore_barrier()

        def body(j, carry):
            pltpu.sync_copy(he.at[pl.ds(wid * NCH * CH + j * CH, CH)], buf)
            pltpu.sync_copy(buf, acc.at[iv.at[j]], add=True)
            return carry

        lax.fori_loop(0, NCH, body, 0)
        plsc.subcore_barrier()
        pltpu.sync_copy(acc.at[pl.ds(sid * ROWS_PER_SUB, ROWS_PER_SUB)],
                        out.at[cid, pl.ds(sid * ROWS_PER_SUB, ROWS_PER_SUB)])

    return k


def _sc_scatter(he, idst, zeros_nd):
    return _sc_scatter_kernel()(he, idst, zeros_nd)


def kernel(x, pe, params, edge_index, batch):
    f32 = jnp.float32
    inp = jnp.concatenate([x, pe], axis=-1)
    x2 = x[:, 2:3]
    src = edge_index[0]
    dst = edge_index[1]
    pad = EP - E
    src_p = jnp.concatenate([src, jnp.zeros((pad,), jnp.int32)])
    dst_p = jnp.concatenate([dst, jnp.zeros((pad,), jnp.int32)])
    dst_s = jnp.concatenate(
        [dst, N + (jnp.arange(pad, dtype=jnp.int32) % NS)])
    isrc = src_p.reshape(NW, NCH, CH)
    idst_g = dst_p.reshape(NW, NCH, CH)
    idst_s = dst_s.reshape(NW, NCH, CH)
    zeros_nd = jnp.zeros((ND, H), f32)

    p = params
    ew1 = jnp.stack([q["W1"].T for q in p["node_enc"]])
    eb1 = jnp.stack([q["b1"][None] for q in p["node_enc"]])
    ew2 = jnp.stack([q["W2"].T for q in p["node_enc"]])
    eb2 = jnp.stack([q["b2"][None] for q in p["node_enc"]])

    eu = p["edge_upd"]
    w1a = [q["W1"][:, :H].T for q in eu]
    w1b = [q["W1"][:, H:2 * H].T for q in eu]
    w1c = [q["W1"][:, 2 * H:].T for q in eu]
    ew2l = [q["W2"].T for q in eu]
    eb1l = [q["b1"][None] for q in eu]
    eb2l = [q["b2"][None] for q in eu]
    nu = p["node_upd"]
    v1a = [q["W1"][:, :H].T for q in nu]
    v1b = [q["W1"][:, H:].T for q in nu]
    nv2 = [q["W2"].T for q in nu]
    nb1 = [q["b1"][None] for q in nu]
    nb2 = [q["b2"][None] for q in nu]
    ebias = p["edge_bias"][None]

    h, ab_tab, tf = _encode(inp, x2, ew1, eb1, ew2, eb2,
                            w1a[0], eb1l[0], w1b[0])

    he = None
    qkv = None
    for l in range(NUM_LAYERS):
        gsrc, gdst = _sc_gather2(ab_tab, isrc, idst_g)
        he = _edge_update(gsrc, gdst, he, w1c[l], ew2l[l], eb2l[l], ebias)
        parts = _sc_scatter(he, idst_s, zeros_nd)
        p0, p1 = parts[0], parts[1]
        if l < NUM_LAYERS - 1:
            h, ab_tab = _node_update(
                h, p0, p1, v1a[l], v1b[l], nb1[l], nv2[l], nb2[l], False,
                w1a[l + 1], eb1l[l + 1], w1b[l + 1])
        else:
            h, qkv = _node_update(
                h, p0, p1, v1a[l], v1b[l], nb1[l], nv2[l], nb2[l], True,
                p["attn_in_W"].T, p["attn_in_b"][None])

    npad = NPAD - N
    batch_p = jnp.concatenate(
        [batch.astype(jnp.int32), jnp.full((npad,), 99, jnp.int32)])
    gid = jnp.arange(G, dtype=batch.dtype)
    starts = jnp.searchsorted(batch, gid, side="left").astype(jnp.int32)
    ends = jnp.searchsorted(batch, gid, side="right").astype(jnp.int32)
    t0 = jnp.arange(NT, dtype=jnp.int32) * TQ
    gmin = jnp.clip(batch_p[t0], 0, G - 1)
    gmax = jnp.clip(batch_p[t0 + TQ - 1], 0, G - 1)
    lo = starts[gmin] // TK
    hi = (ends[gmax] + TK - 1) // TK
    bq = batch_p.astype(f32)[:, None]
    br = batch_p.astype(f32)[None, :]
    qkv_p = jnp.pad(qkv, ((0, npad), (0, 0)))
    h_p = jnp.pad(h, ((0, npad), (0, 0)))
    tf_p = jnp.pad(tf, ((0, npad), (0, 0)), constant_values=99.0)

    out = _attention(
        lo, hi, qkv_p, h_p, tf_p, bq, br,
        p["attn_out_W"].T, p["attn_out_b"][None],
        p["fusion"]["W1"].T, p["fusion"]["b1"][None],
        p["fusion"]["W2"].T, p["fusion"]["b2"][None],
        jnp.stack([q["W1"].T for q in p["dec"]]),
        jnp.stack([q["b1"][None] for q in p["dec"]]),
        jnp.stack([q["W2"].T for q in p["dec"]]),
        jnp.stack([q["b2"][None] for q in p["dec"]]),
    )
    return out[:N]


# speedup vs baseline: 4.7162x; 1.1038x over previous
import functools

import jax
import jax.numpy as jnp
import numpy as np
from jax import lax
from jax.experimental import pallas as pl
from jax.experimental.pallas import tpu as pltpu
from jax.experimental.pallas import tpu_sc as plsc

N = 10000
E = 320000
H = 64
HEADS = 4
DH = H // HEADS
G = 16
OUT_DIM = 128
NUM_LAYERS = 5

NC = 2
NS = 16
NW = NC * NS
CH = 128
NCH = 80
EP = NW * NCH * CH
ND = 10112
ROWS_PER_SUB = ND // NS

BT = 2000
BTE = 4096
NPAD = 10240
TQ = 1024
TK = 1024
NT = NPAD // TQ


def _dot(a, b):
    return jnp.dot(a, b, preferred_element_type=jnp.float32)


def _enc_body(inp_ref, x2_ref, ew1, eb1, ew2, eb2, wa, ba, wb,
              h_ref, ab_ref, tf_ref):
    inp = inp_ref[...]
    t = jnp.clip(jnp.round(x2_ref[...] * 3.0), 1.0, 3.0) - 1.0
    h = jnp.zeros_like(h_ref)
    for i in range(3):
        hid = jnp.maximum(_dot(inp, ew1[i]) + eb1[i], 0.0)
        cand = _dot(hid, ew2[i]) + eb2[i]
        h = jnp.where(t == jnp.float32(i), cand, h)
    h_ref[...] = h
    ab_ref[...] = jnp.concatenate(
        [_dot(h, wa[...]) + ba[...], _dot(h, wb[...])], axis=1)
    tf_ref[...] = t


def _encode(inp, x2, ew1, eb1, ew2, eb2, wa, ba, wb):
    grid = (N // BT,)
    row = lambda i: (i, 0)
    const = lambda shape: pl.BlockSpec(shape, lambda i: (0,) * len(shape))
    return pl.pallas_call(
        _enc_body,
        grid=grid,
        in_specs=[
            pl.BlockSpec((BT, inp.shape[1]), row),
            pl.BlockSpec((BT, 1), row),
            const((3, inp.shape[1], H)), const((3, 1, H)),
            const((3, H, H)), const((3, 1, H)),
            const((H, H)), const((1, H)), const((H, H)),
        ],
        out_specs=[
            pl.BlockSpec((BT, H), row), pl.BlockSpec((BT, 2 * H), row),
            pl.BlockSpec((BT, 1), row),
        ],
        out_shape=[
            jax.ShapeDtypeStruct((N, H), jnp.float32),
            jax.ShapeDtypeStruct((N, 2 * H), jnp.float32),
            jax.ShapeDtypeStruct((N, 1), jnp.float32),
        ],
    )(inp, x2, ew1, eb1, ew2, eb2, wa, ba, wb)


def _edge_first_body(a_ref, b_ref, w1c, w2, b2, eb, o_ref):
    c0 = _dot(eb[...], w1c[...])
    hid = jnp.maximum(a_ref[:, :H] + b_ref[:, H:] + c0, 0.0)
    o_ref[...] = eb[...] + _dot(hid, w2[...]) + b2[...]


def _edge_body(a_ref, b_ref, he_ref, w1c, w2, b2, o_ref):
    he = he_ref[...]
    hid = jnp.maximum(a_ref[:, :H] + b_ref[:, H:] + _dot(he, w1c[...]), 0.0)
    o_ref[...] = he + _dot(hid, w2[...]) + b2[...]


def _edge_update(gsrc, gdst, he, w1c, w2, b2, eb):
    grid = (EP // BTE,)
    row = lambda i: (i, 0)
    const = lambda shape: pl.BlockSpec(shape, lambda i: (0,) * len(shape))
    tile = pl.BlockSpec((BTE, H), row)
    a_spec = pl.BlockSpec((BTE, 2 * H), row)
    b_spec = pl.BlockSpec((BTE, 2 * H), row)
    if he is None:
        return pl.pallas_call(
            _edge_first_body, grid=grid,
            in_specs=[a_spec, b_spec, const((H, H)), const((H, H)),
                      const((1, H)), const((1, H))],
            out_specs=tile,
            out_shape=jax.ShapeDtypeStruct((EP, H), jnp.float32),
        )(gsrc, gdst, w1c, w2, b2, eb)
    return pl.pallas_call(
        _edge_body, grid=grid,
        in_specs=[a_spec, b_spec, tile, const((H, H)), const((H, H)),
                  const((1, H))],
        out_specs=tile,
        out_shape=jax.ShapeDtypeStruct((EP, H), jnp.float32),
    )(gsrc, gdst, he, w1c, w2, b2)


def _node_body(h_ref, p0_ref, p1_ref, v1a, v1b, nb1, v2, nb2, wa, ba, wb,
               h_out, ab_out):
    h = h_ref[...]
    m = p0_ref[...] + p1_ref[...]
    hid = jnp.maximum(_dot(h, v1a[...]) + _dot(m, v1b[...]) + nb1[...], 0.0)
    hn = h + _dot(hid, v2[...]) + nb2[...]
    h_out[...] = hn
    ab_out[...] = jnp.concatenate(
        [_dot(hn, wa[...]) + ba[...], _dot(hn, wb[...])], axis=1)


def _node_last_body(h_ref, p0_ref, p1_ref, v1a, v1b, nb1, v2, nb2, win, bin_,
                    h_out, qkv_out):
    h = h_ref[...]
    m = p0_ref[...] + p1_ref[...]
    hid = jnp.maximum(_dot(h, v1a[...]) + _dot(m, v1b[...]) + nb1[...], 0.0)
    hn = h + _dot(hid, v2[...]) + nb2[...]
    h_out[...] = hn
    qkv_out[...] = _dot(hn, win[...]) + bin_[...]


def _node_update(h, p0, p1, v1a, v1b, nb1, v2, nb2, last, *proj):
    grid = (N // BT,)
    row = lambda i: (i, 0)
    const = lambda shape: pl.BlockSpec(shape, lambda i: (0,) * len(shape))
    tile = pl.BlockSpec((BT, H), row)
    if not last:
        wa, ba, wb = proj
        return pl.pallas_call(
            _node_body, grid=grid,
            in_specs=[tile, tile, tile, const((H, H)), const((H, H)),
                      const((1, H)), const((H, H)), const((1, H)),
                      const((H, H)), const((1, H)), const((H, H))],
            out_specs=[tile, pl.BlockSpec((BT, 2 * H), row)],
            out_shape=[jax.ShapeDtypeStruct((N, H), jnp.float32),
                       jax.ShapeDtypeStruct((N, 2 * H), jnp.float32)],
        )(h, p0, p1, v1a, v1b, nb1, v2, nb2, wa, ba, wb)
    win, bin_ = proj
    return pl.pallas_call(
        _node_last_body, grid=grid,
        in_specs=[tile, tile, tile, const((H, H)), const((H, H)),
                  const((1, H)), const((H, H)), const((1, H)),
                  const((H, 3 * H)), const((1, 3 * H))],
        out_specs=[tile, pl.BlockSpec((BT, 3 * H), row)],
        out_shape=[jax.ShapeDtypeStruct((N, H), jnp.float32),
                   jax.ShapeDtypeStruct((N, 3 * H), jnp.float32)],
    )(h, p0, p1, v1a, v1b, nb1, v2, nb2, win, bin_)


def _attn_body(lo_ref, hi_ref, qkv_ref, h_ref, tf_ref, bq_ref, br_ref,
               wout, bout, wf1, bf1, wf2, bf2, wd1, bd1, wd2, bd2, o_ref):
    t = pl.program_id(0)
    lo = lo_ref[t]
    hi = hi_ref[t]
    bq = bq_ref[...]
    scale = jnp.float32(1.0 / np.sqrt(DH))
    qt = qkv_ref[pl.ds(pl.multiple_of(t * TQ, TQ), TQ), :]
    qs = [qt[:, hd * DH:(hd + 1) * DH] * scale for hd in range(HEADS)]

    def body(j, carry):
        off = pl.multiple_of(j * TK, TK)
        blk = qkv_ref[pl.ds(off, TK), :]
        bk = br_ref[:, pl.ds(off, TK)]
        mask = bq == bk
        new = []
        for hd in range(HEADS):
            m, l_, acc = carry[hd]
            kh = blk[:, H + hd * DH:H + (hd + 1) * DH]
            vh = blk[:, 2 * H + hd * DH:2 * H + (hd + 1) * DH]
            s = lax.dot_general(qs[hd], kh, (((1,), (1,)), ((), ())),
                                preferred_element_type=jnp.float32)
            s = jnp.where(mask, s, jnp.float32(-1e9))
            mn = jnp.maximum(m, jnp.max(s, axis=1, keepdims=True))
            alpha = jnp.exp(m - mn)
            p = jnp.exp(s - mn)
            l2 = l_ * alpha + jnp.sum(p, axis=1, keepdims=True)
            acc2 = acc * alpha + _dot(p, vh)
            new.append((mn, l2, acc2))
        return tuple(new)

    init = tuple((jnp.full((TQ, 1), -1e30, jnp.float32),
                  jnp.zeros((TQ, 1), jnp.float32),
                  jnp.zeros((TQ, DH), jnp.float32)) for _ in range(HEADS))
    carry = lax.fori_loop(lo, hi, body, init)
    o = jnp.concatenate([acc / l_ for (m, l_, acc) in carry], axis=1)
    h = h_ref[...]
    ao = _dot(o, wout[...]) + bout[...]
    z = h + h + ao
    hf = _dot(jnp.maximum(_dot(z, wf1[...]) + bf1[...], 0.0), wf2[...]) + bf2[...]
    tval = tf_ref[...]
    res = jnp.zeros_like(o_ref)
    for i in range(3):
        cand = _dot(jnp.maximum(_dot(hf, wd1[i]) + bd1[i], 0.0), wd2[i]) + bd2[i]
        res = jnp.where(tval == jnp.float32(i), cand, res)
    o_ref[...] = res


def _attention(lo, hi, qkv, h, tf, bq, br, wout, bout, wf1, bf1, wf2, bf2,
               wd1, bd1, wd2, bd2):
    row = lambda i, *_: (i, 0)
    const = lambda shape: pl.BlockSpec(shape, lambda i, *_: (0,) * len(shape))
    grid_spec = pltpu.PrefetchScalarGridSpec(
        num_scalar_prefetch=2,
        grid=(NT,),
        in_specs=[
            const((NPAD, 3 * H)),
            pl.BlockSpec((TQ, H), row),
            pl.BlockSpec((TQ, 1), row),
            pl.BlockSpec((TQ, 1), row),
            const((1, NPAD)),
            const((H, H)), const((1, H)),
            const((H, H)), const((1, H)),
            const((H, H)), const((1, H)),
            const((3, H, H)), const((3, 1, H)),
            const((3, H, OUT_DIM)), const((3, 1, OUT_DIM)),
        ],
        out_specs=pl.BlockSpec((TQ, OUT_DIM), row),
    )
    return pl.pallas_call(
        _attn_body,
        grid_spec=grid_spec,
        out_shape=jax.ShapeDtypeStruct((NPAD, OUT_DIM), jnp.float32),
    )(lo, hi, qkv, h, tf, bq, br, wout, bout, wf1, bf1, wf2, bf2,
      wd1, bd1, wd2, bd2)


@functools.lru_cache(maxsize=1)
def _sc_gather2_kernel():
    mesh = plsc.VectorSubcoreMesh(core_axis_name="c", subcore_axis_name="s")

    NB = 2

    @functools.partial(
        pl.kernel,
        mesh=mesh,
        out_type=[jax.ShapeDtypeStruct((EP, 2 * H), jnp.float32),
                  jax.ShapeDtypeStruct((EP, 2 * H), jnp.float32)],
        scratch_types=[
            pltpu.VMEM((NCH, CH), jnp.int32),
            pltpu.VMEM((NCH, CH), jnp.int32),
            pltpu.VMEM((NB, CH, 2 * H), jnp.float32),
            pltpu.VMEM((NB, CH, 2 * H), jnp.float32),
            pltpu.SemaphoreType.DMA,
            pltpu.SemaphoreType.DMA,
            pltpu.SemaphoreType.DMA,
            pltpu.SemaphoreType.DMA,
        ],
    )
    def k(tab, isrc, idst, oa, ob, iv_s, iv_d, bufa, bufb, sa0, sa1, sb0, sb1):
        wid = lax.axis_index("s") * NC + lax.axis_index("c")
        base = wid * (NCH * CH)
        pltpu.sync_copy(isrc.at[wid], iv_s)
        pltpu.sync_copy(idst.at[wid], iv_d)
        sas = (sa0, sa1)
        sbs = (sb0, sb1)

        for b in range(NB):
            pltpu.async_copy(tab.at[iv_s.at[b]], bufa.at[b], sas[b])
            pltpu.async_copy(tab.at[iv_d.at[b]], bufb.at[b], sbs[b])

        def body(t, carry):
            for b in range(NB):
                j = t * NB + b
                pltpu.make_async_copy(tab.at[iv_s.at[j]], bufa.at[b],
                                      sas[b]).wait()
                pltpu.make_async_copy(tab.at[iv_d.at[j]], bufb.at[b],
                                      sbs[b]).wait()
                pltpu.sync_copy(bufa.at[b], oa.at[pl.ds(base + j * CH, CH)])
                pltpu.sync_copy(bufb.at[b], ob.at[pl.ds(base + j * CH, CH)])

                @pl.when(j + NB < NCH)
                def _():
                    pltpu.async_copy(tab.at[iv_s.at[j + NB]], bufa.at[b],
                                     sas[b])
                    pltpu.async_copy(tab.at[iv_d.at[j + NB]], bufb.at[b],
                                     sbs[b])
            return carry

        lax.fori_loop(0, NCH // NB, body, 0)

    return k


def _sc_gather2(tab, isrc, idst):
    return _sc_gather2_kernel()(tab, isrc, idst)


@functools.lru_cache(maxsize=1)
def _sc_scatter_kernel():
    mesh = plsc.VectorSubcoreMesh(core_axis_name="c", subcore_axis_name="s")

    NB = 2

    @functools.partial(
        pl.kernel,
        mesh=mesh,
        out_type=jax.ShapeDtypeStruct((NC, ND, H), jnp.float32),
        scratch_types=[
            pltpu.VMEM((NCH, CH), jnp.int32),
            pltpu.VMEM((NB, CH, H), jnp.float32),
            pltpu.VMEM_SHARED((ND, H), jnp.float32),
            pltpu.SemaphoreType.DMA,
            pltpu.SemaphoreType.DMA,
        ],
    )
    def k(he, idst, zeros_nd, out, iv, buf, acc, s0, s1):
        cid = lax.axis_index("c")
        sid = lax.axis_index("s")
        wid = sid * NC + cid
        ebase = wid * NCH * CH
        pltpu.sync_copy(zeros_nd.at[pl.ds(sid * ROWS_PER_SUB, ROWS_PER_SUB)],
                        acc.at[pl.ds(sid * ROWS_PER_SUB, ROWS_PER_SUB)])
        pltpu.sync_copy(idst.at[wid], iv)
        plsc.subcore_barrier()
        sems = (s0, s1)

        for b in range(NB):
            pltpu.async_copy(he.at[pl.ds(ebase + b * CH, CH)], buf.at[b],
                             sems[b])

        def body(t, carry):
            for b in range(NB):
                j = t * NB + b
                pltpu.make_async_copy(he.at[pl.ds(ebase + j * CH, CH)],
                                      buf.at[b], sems[b]).wait()
                pltpu.sync_copy(buf.at[b], acc.at[iv.at[j]], add=True)

                @pl.when(j + NB < NCH)
                def _():
                    pltpu.async_copy(he.at[pl.ds(ebase + (j + NB) * CH, CH)],
                                     buf.at[b], sems[b])
            return carry

        lax.fori_loop(0, NCH // NB, body, 0)
        plsc.subcore_barrier()
        pltpu.sync_copy(acc.at[pl.ds(sid * ROWS_PER_SUB, ROWS_PER_SUB)],
                        out.at[cid, pl.ds(sid * ROWS_PER_SUB, ROWS_PER_SUB)])

    return k


def _sc_scatter(he, idst, zeros_nd):
    return _sc_scatter_kernel()(he, idst, zeros_nd)


def kernel(x, pe, params, edge_index, batch):
    f32 = jnp.float32
    inp = jnp.concatenate([x, pe], axis=-1)
    x2 = x[:, 2:3]
    src = edge_index[0]
    dst = edge_index[1]
    pad = EP - E
    src_p = jnp.concatenate([src, jnp.zeros((pad,), jnp.int32)])
    dst_p = jnp.concatenate([dst, jnp.zeros((pad,), jnp.int32)])
    dst_s = jnp.concatenate(
        [dst, N + (jnp.arange(pad, dtype=jnp.int32) % NS)])
    isrc = src_p.reshape(NW, NCH, CH)
    idst_g = dst_p.reshape(NW, NCH, CH)
    idst_s = dst_s.reshape(NW, NCH, CH)
    zeros_nd = jnp.zeros((ND, H), f32)

    p = params
    ew1 = jnp.stack([q["W1"].T for q in p["node_enc"]])
    eb1 = jnp.stack([q["b1"][None] for q in p["node_enc"]])
    ew2 = jnp.stack([q["W2"].T for q in p["node_enc"]])
    eb2 = jnp.stack([q["b2"][None] for q in p["node_enc"]])

    eu = p["edge_upd"]
    w1a = [q["W1"][:, :H].T for q in eu]
    w1b = [q["W1"][:, H:2 * H].T for q in eu]
    w1c = [q["W1"][:, 2 * H:].T for q in eu]
    ew2l = [q["W2"].T for q in eu]
    eb1l = [q["b1"][None] for q in eu]
    eb2l = [q["b2"][None] for q in eu]
    nu = p["node_upd"]
    v1a = [q["W1"][:, :H].T for q in nu]
    v1b = [q["W1"][:, H:].T for q in nu]
    nv2 = [q["W2"].T for q in nu]
    nb1 = [q["b1"][None] for q in nu]
    nb2 = [q["b2"][None] for q in nu]
    ebias = p["edge_bias"][None]

    h, ab_tab, tf = _encode(inp, x2, ew1, eb1, ew2, eb2,
                            w1a[0], eb1l[0], w1b[0])

    he = None
    qkv = None
    for l in range(NUM_LAYERS):
        gsrc, gdst = _sc_gather2(ab_tab, isrc, idst_g)
        he = _edge_update(gsrc, gdst, he, w1c[l], ew2l[l], eb2l[l], ebias)
        parts = _sc_scatter(he, idst_s, zeros_nd)
        p0, p1 = parts[0], parts[1]
        if l < NUM_LAYERS - 1:
            h, ab_tab = _node_update(
                h, p0, p1, v1a[l], v1b[l], nb1[l], nv2[l], nb2[l], False,
                w1a[l + 1], eb1l[l + 1], w1b[l + 1])
        else:
            h, qkv = _node_update(
                h, p0, p1, v1a[l], v1b[l], nb1[l], nv2[l], nb2[l], True,
                p["attn_in_W"].T, p["attn_in_b"][None])

    npad = NPAD - N
    batch_p = jnp.concatenate(
        [batch.astype(jnp.int32), jnp.full((npad,), 99, jnp.int32)])
    gid = jnp.arange(G, dtype=batch.dtype)
    starts = jnp.searchsorted(batch, gid, side="left").astype(jnp.int32)
    ends = jnp.searchsorted(batch, gid, side="right").astype(jnp.int32)
    t0 = jnp.arange(NT, dtype=jnp.int32) * TQ
    gmin = jnp.clip(batch_p[t0], 0, G - 1)
    gmax = jnp.clip(batch_p[t0 + TQ - 1], 0, G - 1)
    lo = starts[gmin] // TK
    hi = (ends[gmax] + TK - 1) // TK
    bq = batch_p.astype(f32)[:, None]
    br = batch_p.astype(f32)[None, :]
    qkv_p = jnp.pad(qkv, ((0, npad), (0, 0)))
    h_p = jnp.pad(h, ((0, npad), (0, 0)))
    tf_p = jnp.pad(tf, ((0, npad), (0, 0)), constant_values=99.0)

    out = _attention(
        lo, hi, qkv_p, h_p, tf_p, bq, br,
        p["attn_out_W"].T, p["attn_out_b"][None],
        p["fusion"]["W1"].T, p["fusion"]["b1"][None],
        p["fusion"]["W2"].T, p["fusion"]["b2"][None],
        jnp.stack([q["W1"].T for q in p["dec"]]),
        jnp.stack([q["b1"][None] for q in p["dec"]]),
        jnp.stack([q["W2"].T for q in p["dec"]]),
        jnp.stack([q["b2"][None] for q in p["dec"]]),
    )
    return out[:N]


# speedup vs baseline: 5.4616x; 1.1581x over previous
import functools

import jax
import jax.numpy as jnp
import numpy as np
from jax import lax
from jax.experimental import pallas as pl
from jax.experimental.pallas import tpu as pltpu
from jax.experimental.pallas import tpu_sc as plsc

N = 10000
E = 320000
H = 64
HEADS = 4
DH = H // HEADS
G = 16
OUT_DIM = 128
NUM_LAYERS = 5

NC = 2
NS = 16
NW = NC * NS
CH = 64
NCH = 160
EP = NW * NCH * CH
ND = 10112
ROWS_PER_SUB = ND // NS

BT = 2000
BTE = 4096
NPAD = 10240
TQ = 1024
TK = 1024
NT = NPAD // TQ


def _dot(a, b):
    return jnp.dot(a, b, preferred_element_type=jnp.float32)


def _enc_body(inp_ref, x2_ref, ew1, eb1, ew2, eb2, wa, ba, wb,
              h_ref, ab_ref, tf_ref):
    inp = inp_ref[...]
    t = jnp.clip(jnp.round(x2_ref[...] * 3.0), 1.0, 3.0) - 1.0
    h = jnp.zeros_like(h_ref)
    for i in range(3):
        hid = jnp.maximum(_dot(inp, ew1[i]) + eb1[i], 0.0)
        cand = _dot(hid, ew2[i]) + eb2[i]
        h = jnp.where(t == jnp.float32(i), cand, h)
    h_ref[...] = h
    ab_ref[...] = jnp.concatenate(
        [_dot(h, wa[...]) + ba[...], _dot(h, wb[...])], axis=1)
    tf_ref[...] = t


def _encode(inp, x2, ew1, eb1, ew2, eb2, wa, ba, wb):
    grid = (N // BT,)
    row = lambda i: (i, 0)
    const = lambda shape: pl.BlockSpec(shape, lambda i: (0,) * len(shape))
    return pl.pallas_call(
        _enc_body,
        grid=grid,
        in_specs=[
            pl.BlockSpec((BT, inp.shape[1]), row),
            pl.BlockSpec((BT, 1), row),
            const((3, inp.shape[1], H)), const((3, 1, H)),
            const((3, H, H)), const((3, 1, H)),
            const((H, H)), const((1, H)), const((H, H)),
        ],
        out_specs=[
            pl.BlockSpec((BT, H), row), pl.BlockSpec((BT, 2 * H), row),
            pl.BlockSpec((BT, 1), row),
        ],
        out_shape=[
            jax.ShapeDtypeStruct((N, H), jnp.float32),
            jax.ShapeDtypeStruct((N, 2 * H), jnp.float32),
            jax.ShapeDtypeStruct((N, 1), jnp.float32),
        ],
    )(inp, x2, ew1, eb1, ew2, eb2, wa, ba, wb)


def _edge_first_body(a_ref, b_ref, w1c, w2, b2, eb, o_ref):
    c0 = _dot(eb[...], w1c[...])
    hid = jnp.maximum(a_ref[:, :H] + b_ref[:, H:] + c0, 0.0)
    o_ref[...] = eb[...] + _dot(hid, w2[...]) + b2[...]


def _edge_body(a_ref, b_ref, he_ref, w1c, w2, b2, o_ref):
    he = he_ref[...]
    hid = jnp.maximum(a_ref[:, :H] + b_ref[:, H:] + _dot(he, w1c[...]), 0.0)
    o_ref[...] = he + _dot(hid, w2[...]) + b2[...]


def _edge_update(gsrc, gdst, he, w1c, w2, b2, eb):
    grid = (EP // BTE,)
    row = lambda i: (i, 0)
    const = lambda shape: pl.BlockSpec(shape, lambda i: (0,) * len(shape))
    tile = pl.BlockSpec((BTE, H), row)
    a_spec = pl.BlockSpec((BTE, 2 * H), row)
    b_spec = pl.BlockSpec((BTE, 2 * H), row)
    if he is None:
        return pl.pallas_call(
            _edge_first_body, grid=grid,
            in_specs=[a_spec, b_spec, const((H, H)), const((H, H)),
                      const((1, H)), const((1, H))],
            out_specs=tile,
            out_shape=jax.ShapeDtypeStruct((EP, H), jnp.float32),
        )(gsrc, gdst, w1c, w2, b2, eb)
    return pl.pallas_call(
        _edge_body, grid=grid,
        in_specs=[a_spec, b_spec, tile, const((H, H)), const((H, H)),
                  const((1, H))],
        out_specs=tile,
        out_shape=jax.ShapeDtypeStruct((EP, H), jnp.float32),
    )(gsrc, gdst, he, w1c, w2, b2)


def _node_body(h_ref, p0_ref, p1_ref, v1a, v1b, nb1, v2, nb2, wa, ba, wb,
               h_out, ab_out):
    h = h_ref[...]
    m = p0_ref[...] + p1_ref[...]
    hid = jnp.maximum(_dot(h, v1a[...]) + _dot(m, v1b[...]) + nb1[...], 0.0)
    hn = h + _dot(hid, v2[...]) + nb2[...]
    h_out[...] = hn
    ab_out[...] = jnp.concatenate(
        [_dot(hn, wa[...]) + ba[...], _dot(hn, wb[...])], axis=1)


def _node_last_body(h_ref, p0_ref, p1_ref, v1a, v1b, nb1, v2, nb2, win, bin_,
                    h_out, qkv_out):
    h = h_ref[...]
    m = p0_ref[...] + p1_ref[...]
    hid = jnp.maximum(_dot(h, v1a[...]) + _dot(m, v1b[...]) + nb1[...], 0.0)
    hn = h + _dot(hid, v2[...]) + nb2[...]
    h_out[...] = hn
    qkv_out[...] = _dot(hn, win[...]) + bin_[...]


def _node_update(h, p0, p1, v1a, v1b, nb1, v2, nb2, last, *proj):
    grid = (N // BT,)
    row = lambda i: (i, 0)
    const = lambda shape: pl.BlockSpec(shape, lambda i: (0,) * len(shape))
    tile = pl.BlockSpec((BT, H), row)
    if not last:
        wa, ba, wb = proj
        return pl.pallas_call(
            _node_body, grid=grid,
            in_specs=[tile, tile, tile, const((H, H)), const((H, H)),
                      const((1, H)), const((H, H)), const((1, H)),
                      const((H, H)), const((1, H)), const((H, H))],
            out_specs=[tile, pl.BlockSpec((BT, 2 * H), row)],
            out_shape=[jax.ShapeDtypeStruct((N, H), jnp.float32),
                       jax.ShapeDtypeStruct((N, 2 * H), jnp.float32)],
        )(h, p0, p1, v1a, v1b, nb1, v2, nb2, wa, ba, wb)
    win, bin_ = proj
    return pl.pallas_call(
        _node_last_body, grid=grid,
        in_specs=[tile, tile, tile, const((H, H)), const((H, H)),
                  const((1, H)), const((H, H)), const((1, H)),
                  const((H, 3 * H)), const((1, 3 * H))],
        out_specs=[tile, pl.BlockSpec((BT, 3 * H), row)],
        out_shape=[jax.ShapeDtypeStruct((N, H), jnp.float32),
                   jax.ShapeDtypeStruct((N, 3 * H), jnp.float32)],
    )(h, p0, p1, v1a, v1b, nb1, v2, nb2, win, bin_)


def _attn_body(lo_ref, hi_ref, qkv_ref, h_ref, tf_ref, bq_ref, br_ref,
               wout, bout, wf1, bf1, wf2, bf2, wd1, bd1, wd2, bd2, o_ref):
    t = pl.program_id(0)
    lo = lo_ref[t]
    hi = hi_ref[t]
    bq = bq_ref[...]
    scale = jnp.float32(1.0 / np.sqrt(DH))
    qt = qkv_ref[pl.ds(pl.multiple_of(t * TQ, TQ), TQ), :]
    qs = [qt[:, hd * DH:(hd + 1) * DH] * scale for hd in range(HEADS)]

    def body(j, carry):
        off = pl.multiple_of(j * TK, TK)
        blk = qkv_ref[pl.ds(off, TK), :]
        bk = br_ref[:, pl.ds(off, TK)]
        mask = bq == bk
        new = []
        for hd in range(HEADS):
            m, l_, acc = carry[hd]
            kh = blk[:, H + hd * DH:H + (hd + 1) * DH]
            vh = blk[:, 2 * H + hd * DH:2 * H + (hd + 1) * DH]
            s = lax.dot_general(qs[hd], kh, (((1,), (1,)), ((), ())),
                                preferred_element_type=jnp.float32)
            s = jnp.where(mask, s, jnp.float32(-1e9))
            mn = jnp.maximum(m, jnp.max(s, axis=1, keepdims=True))
            alpha = jnp.exp(m - mn)
            p = jnp.exp(s - mn)
            l2 = l_ * alpha + jnp.sum(p, axis=1, keepdims=True)
            acc2 = acc * alpha + _dot(p, vh)
            new.append((mn, l2, acc2))
        return tuple(new)

    init = tuple((jnp.full((TQ, 1), -1e30, jnp.float32),
                  jnp.zeros((TQ, 1), jnp.float32),
                  jnp.zeros((TQ, DH), jnp.float32)) for _ in range(HEADS))
    carry = lax.fori_loop(lo, hi, body, init)
    o = jnp.concatenate([acc / l_ for (m, l_, acc) in carry], axis=1)
    h = h_ref[...]
    ao = _dot(o, wout[...]) + bout[...]
    z = h + h + ao
    hf = _dot(jnp.maximum(_dot(z, wf1[...]) + bf1[...], 0.0), wf2[...]) + bf2[...]
    tval = tf_ref[...]
    res = jnp.zeros_like(o_ref)
    for i in range(3):
        cand = _dot(jnp.maximum(_dot(hf, wd1[i]) + bd1[i], 0.0), wd2[i]) + bd2[i]
        res = jnp.where(tval == jnp.float32(i), cand, res)
    o_ref[...] = res


def _attention(lo, hi, qkv, h, tf, bq, br, wout, bout, wf1, bf1, wf2, bf2,
               wd1, bd1, wd2, bd2):
    row = lambda i, *_: (i, 0)
    const = lambda shape: pl.BlockSpec(shape, lambda i, *_: (0,) * len(shape))
    grid_spec = pltpu.PrefetchScalarGridSpec(
        num_scalar_prefetch=2,
        grid=(NT,),
        in_specs=[
            const((NPAD, 3 * H)),
            pl.BlockSpec((TQ, H), row),
            pl.BlockSpec((TQ, 1), row),
            pl.BlockSpec((TQ, 1), row),
            const((1, NPAD)),
            const((H, H)), const((1, H)),
            const((H, H)), const((1, H)),
            const((H, H)), const((1, H)),
            const((3, H, H)), const((3, 1, H)),
            const((3, H, OUT_DIM)), const((3, 1, OUT_DIM)),
        ],
        out_specs=pl.BlockSpec((TQ, OUT_DIM), row),
    )
    return pl.pallas_call(
        _attn_body,
        grid_spec=grid_spec,
        out_shape=jax.ShapeDtypeStruct((NPAD, OUT_DIM), jnp.float32),
    )(lo, hi, qkv, h, tf, bq, br, wout, bout, wf1, bf1, wf2, bf2,
      wd1, bd1, wd2, bd2)


@functools.lru_cache(maxsize=1)
def _sc_gather_kernel():
    mesh = plsc.VectorSubcoreMesh(core_axis_name="c", subcore_axis_name="s")

    NB = 4

    @functools.partial(
        pl.kernel,
        mesh=mesh,
        out_type=[jax.ShapeDtypeStruct((EP, 2 * H), jnp.float32),
                  jax.ShapeDtypeStruct((EP, 2 * H), jnp.float32)],
        scratch_types=[
            pltpu.VMEM((NCH, CH), jnp.int32),
            pltpu.VMEM((NCH, CH), jnp.int32),
            pltpu.VMEM((NB, CH, 2 * H), jnp.float32),
            pltpu.VMEM((NB, CH, 2 * H), jnp.float32),
        ] + [pltpu.SemaphoreType.DMA] * (2 * NB),
    )
    def k(tab, isrc, idst, oa, ob, iv_s, iv_d, bufa, bufb, *sems):
        wid = lax.axis_index("s") * NC + lax.axis_index("c")
        base = wid * (NCH * CH)
        pltpu.sync_copy(isrc.at[wid], iv_s)
        pltpu.sync_copy(idst.at[wid], iv_d)
        sas = sems[:NB]
        sbs = sems[NB:]

        for b in range(NB):
            pltpu.async_copy(tab.at[iv_s.at[b]], bufa.at[b], sas[b])
            pltpu.async_copy(tab.at[iv_d.at[b]], bufb.at[b], sbs[b])

        def body(t, carry):
            for b in range(NB):
                j = t * NB + b
                pltpu.make_async_copy(tab.at[iv_s.at[j]], bufa.at[b],
                                      sas[b]).wait()
                pltpu.make_async_copy(tab.at[iv_d.at[j]], bufb.at[b],
                                      sbs[b]).wait()
                pltpu.sync_copy(bufa.at[b], oa.at[pl.ds(base + j * CH, CH)])
                pltpu.sync_copy(bufb.at[b], ob.at[pl.ds(base + j * CH, CH)])

                @pl.when(j + NB < NCH)
                def _():
                    pltpu.async_copy(tab.at[iv_s.at[j + NB]], bufa.at[b],
                                     sas[b])
                    pltpu.async_copy(tab.at[iv_d.at[j + NB]], bufb.at[b],
                                     sbs[b])
            return carry

        lax.fori_loop(0, NCH // NB, body, 0)

    return k


def _sc_gather2(tab, isrc, idst):
    return _sc_gather_kernel()(tab, isrc, idst)


@functools.lru_cache(maxsize=1)
def _sc_scatter_kernel():
    mesh = plsc.VectorSubcoreMesh(core_axis_name="c", subcore_axis_name="s")

    NB = 2

    @functools.partial(
        pl.kernel,
        mesh=mesh,
        out_type=jax.ShapeDtypeStruct((NC, ND, H), jnp.float32),
        scratch_types=[
            pltpu.VMEM((NCH, CH), jnp.int32),
            pltpu.VMEM((NB, CH, H), jnp.float32),
            pltpu.VMEM_SHARED((ND, H), jnp.float32),
            pltpu.SemaphoreType.DMA,
            pltpu.SemaphoreType.DMA,
        ],
    )
    def k(he, idst, zeros_nd, out, iv, buf, acc, s0, s1):
        cid = lax.axis_index("c")
        sid = lax.axis_index("s")
        wid = sid * NC + cid
        ebase = wid * NCH * CH
        pltpu.sync_copy(zeros_nd.at[pl.ds(sid * ROWS_PER_SUB, ROWS_PER_SUB)],
                        acc.at[pl.ds(sid * ROWS_PER_SUB, ROWS_PER_SUB)])
        pltpu.sync_copy(idst.at[wid], iv)
        plsc.subcore_barrier()
        sems = (s0, s1)

        for b in range(NB):
            pltpu.async_copy(he.at[pl.ds(ebase + b * CH, CH)], buf.at[b],
                             sems[b])

        def body(t, carry):
            for b in range(NB):
                j = t * NB + b
                pltpu.make_async_copy(he.at[pl.ds(ebase + j * CH, CH)],
                                      buf.at[b], sems[b]).wait()
                pltpu.sync_copy(buf.at[b], acc.at[iv.at[j]], add=True)

                @pl.when(j + NB < NCH)
                def _():
                    pltpu.async_copy(he.at[pl.ds(ebase + (j + NB) * CH, CH)],
                                     buf.at[b], sems[b])
            return carry

        lax.fori_loop(0, NCH // NB, body, 0)
        plsc.subcore_barrier()
        pltpu.sync_copy(acc.at[pl.ds(sid * ROWS_PER_SUB, ROWS_PER_SUB)],
                        out.at[cid, pl.ds(sid * ROWS_PER_SUB, ROWS_PER_SUB)])

    return k


def _sc_scatter(he, idst, zeros_nd):
    return _sc_scatter_kernel()(he, idst, zeros_nd)


def kernel(x, pe, params, edge_index, batch):
    f32 = jnp.float32
    inp = jnp.concatenate([x, pe], axis=-1)
    x2 = x[:, 2:3]
    src = edge_index[0]
    dst = edge_index[1]
    pad = EP - E
    src_p = jnp.concatenate([src, jnp.zeros((pad,), jnp.int32)])
    dst_p = jnp.concatenate([dst, jnp.zeros((pad,), jnp.int32)])
    dst_s = jnp.concatenate(
        [dst, N + (jnp.arange(pad, dtype=jnp.int32) % NS)])
    isrc = src_p.reshape(NW, NCH, CH)
    idst_g = dst_p.reshape(NW, NCH, CH)
    idst_s = dst_s.reshape(NW, NCH, CH)
    zeros_nd = jnp.zeros((ND, H), f32)

    p = params
    ew1 = jnp.stack([q["W1"].T for q in p["node_enc"]])
    eb1 = jnp.stack([q["b1"][None] for q in p["node_enc"]])
    ew2 = jnp.stack([q["W2"].T for q in p["node_enc"]])
    eb2 = jnp.stack([q["b2"][None] for q in p["node_enc"]])

    eu = p["edge_upd"]
    w1a = [q["W1"][:, :H].T for q in eu]
    w1b = [q["W1"][:, H:2 * H].T for q in eu]
    w1c = [q["W1"][:, 2 * H:].T for q in eu]
    ew2l = [q["W2"].T for q in eu]
    eb1l = [q["b1"][None] for q in eu]
    eb2l = [q["b2"][None] for q in eu]
    nu = p["node_upd"]
    v1a = [q["W1"][:, :H].T for q in nu]
    v1b = [q["W1"][:, H:].T for q in nu]
    nv2 = [q["W2"].T for q in nu]
    nb1 = [q["b1"][None] for q in nu]
    nb2 = [q["b2"][None] for q in nu]
    ebias = p["edge_bias"][None]

    h, ab_tab, tf = _encode(inp, x2, ew1, eb1, ew2, eb2,
                            w1a[0], eb1l[0], w1b[0])

    he = None
    qkv = None
    for l in range(NUM_LAYERS):
        gsrc, gdst = _sc_gather2(ab_tab, isrc, idst_g)
        he = _edge_update(gsrc, gdst, he, w1c[l], ew2l[l], eb2l[l], ebias)
        parts = _sc_scatter(he, idst_s, zeros_nd)
        p0, p1 = parts[0], parts[1]
        if l < NUM_LAYERS - 1:
            h, ab_tab = _node_update(
                h, p0, p1, v1a[l], v1b[l], nb1[l], nv2[l], nb2[l], False,
                w1a[l + 1], eb1l[l + 1], w1b[l + 1])
        else:
            h, qkv = _node_update(
                h, p0, p1, v1a[l], v1b[l], nb1[l], nv2[l], nb2[l], True,
                p["attn_in_W"].T, p["attn_in_b"][None])

    npad = NPAD - N
    batch_p = jnp.concatenate(
        [batch.astype(jnp.int32), jnp.full((npad,), 99, jnp.int32)])
    gid = jnp.arange(G, dtype=batch.dtype)
    starts = jnp.searchsorted(batch, gid, side="left").astype(jnp.int32)
    ends = jnp.searchsorted(batch, gid, side="right").astype(jnp.int32)
    t0 = jnp.arange(NT, dtype=jnp.int32) * TQ
    gmin = jnp.clip(batch_p[t0], 0, G - 1)
    gmax = jnp.clip(batch_p[t0 + TQ - 1], 0, G - 1)
    lo = starts[gmin] // TK
    hi = (ends[gmax] + TK - 1) // TK
    bq = batch_p.astype(f32)[:, None]
    br = batch_p.astype(f32)[None, :]
    qkv_p = jnp.pad(qkv, ((0, npad), (0, 0)))
    h_p = jnp.pad(h, ((0, npad), (0, 0)))
    tf_p = jnp.pad(tf, ((0, npad), (0, 0)), constant_values=99.0)

    out = _attention(
        lo, hi, qkv_p, h_p, tf_p, bq, br,
        p["attn_out_W"].T, p["attn_out_b"][None],
        p["fusion"]["W1"].T, p["fusion"]["b1"][None],
        p["fusion"]["W2"].T, p["fusion"]["b2"][None],
        jnp.stack([q["W1"].T for q in p["dec"]]),
        jnp.stack([q["b1"][None] for q in p["dec"]]),
        jnp.stack([q["W2"].T for q in p["dec"]]),
        jnp.stack([q["b2"][None] for q in p["dec"]]),
    )
    return out[:N]


# speedup vs baseline: 5.4625x; 1.0002x over previous
import functools

import jax
import jax.numpy as jnp
import numpy as np
from jax import lax
from jax.experimental import pallas as pl
from jax.experimental.pallas import tpu as pltpu
from jax.experimental.pallas import tpu_sc as plsc

N = 10000
E = 320000
H = 64
HEADS = 4
DH = H // HEADS
G = 16
OUT_DIM = 128
NUM_LAYERS = 5

NC = 2
NS = 16
NW = NC * NS
CH = 64
NCH = 160
EP = NW * NCH * CH
ND = 10112
ROWS_PER_SUB = ND // NS

BT = 2000
BTE = 4096
NPAD = 10240
TQ = 1024
TK = 1024
NT = NPAD // TQ


def _dot(a, b):
    return jnp.dot(a, b, preferred_element_type=jnp.float32)


def _enc_body(inp_ref, x2_ref, ew1, eb1, ew2, eb2, wa, ba, wb,
              h_ref, ab_ref, tf_ref):
    inp = inp_ref[...]
    t = jnp.clip(jnp.round(x2_ref[...] * 3.0), 1.0, 3.0) - 1.0
    h = jnp.zeros_like(h_ref)
    for i in range(3):
        hid = jnp.maximum(_dot(inp, ew1[i]) + eb1[i], 0.0)
        cand = _dot(hid, ew2[i]) + eb2[i]
        h = jnp.where(t == jnp.float32(i), cand, h)
    h_ref[...] = h
    ab_ref[...] = jnp.concatenate(
        [_dot(h, wa[...]) + ba[...], _dot(h, wb[...])], axis=1)
    tf_ref[...] = t


def _encode(inp, x2, ew1, eb1, ew2, eb2, wa, ba, wb):
    grid = (N // BT,)
    row = lambda i: (i, 0)
    const = lambda shape: pl.BlockSpec(shape, lambda i: (0,) * len(shape))
    return pl.pallas_call(
        _enc_body,
        grid=grid,
        in_specs=[
            pl.BlockSpec((BT, inp.shape[1]), row),
            pl.BlockSpec((BT, 1), row),
            const((3, inp.shape[1], H)), const((3, 1, H)),
            const((3, H, H)), const((3, 1, H)),
            const((H, H)), const((1, H)), const((H, H)),
        ],
        out_specs=[
            pl.BlockSpec((BT, H), row), pl.BlockSpec((BT, 2 * H), row),
            pl.BlockSpec((BT, 1), row),
        ],
        out_shape=[
            jax.ShapeDtypeStruct((N, H), jnp.float32),
            jax.ShapeDtypeStruct((N, 2 * H), jnp.float32),
            jax.ShapeDtypeStruct((N, 1), jnp.float32),
        ],
    )(inp, x2, ew1, eb1, ew2, eb2, wa, ba, wb)


def _edge_first_body(a_ref, b_ref, w1c, w2, b2, eb, o_ref):
    c0 = _dot(eb[...], w1c[...])
    hid = jnp.maximum(a_ref[:, :H] + b_ref[:, H:] + c0, 0.0)
    o_ref[...] = eb[...] + _dot(hid, w2[...]) + b2[...]


def _edge_body(a_ref, b_ref, he_ref, w1c, w2, b2, o_ref):
    he = he_ref[...]
    hid = jnp.maximum(a_ref[:, :H] + b_ref[:, H:] + _dot(he, w1c[...]), 0.0)
    o_ref[...] = he + _dot(hid, w2[...]) + b2[...]


def _edge_update(gsrc, gdst, he, w1c, w2, b2, eb):
    grid = (EP // BTE,)
    row = lambda i: (i, 0)
    const = lambda shape: pl.BlockSpec(shape, lambda i: (0,) * len(shape))
    tile = pl.BlockSpec((BTE, H), row)
    a_spec = pl.BlockSpec((BTE, 2 * H), row)
    b_spec = pl.BlockSpec((BTE, 2 * H), row)
    if he is None:
        return pl.pallas_call(
            _edge_first_body, grid=grid,
            in_specs=[a_spec, b_spec, const((H, H)), const((H, H)),
                      const((1, H)), const((1, H))],
            out_specs=tile,
            out_shape=jax.ShapeDtypeStruct((EP, H), jnp.float32),
        )(gsrc, gdst, w1c, w2, b2, eb)
    return pl.pallas_call(
        _edge_body, grid=grid,
        in_specs=[a_spec, b_spec, tile, const((H, H)), const((H, H)),
                  const((1, H))],
        out_specs=tile,
        out_shape=jax.ShapeDtypeStruct((EP, H), jnp.float32),
    )(gsrc, gdst, he, w1c, w2, b2)


def _node_body(h_ref, p0_ref, p1_ref, v1a, v1b, nb1, v2, nb2, wa, ba, wb,
               h_out, ab_out):
    h = h_ref[...]
    m = p0_ref[...] + p1_ref[...]
    hid = jnp.maximum(_dot(h, v1a[...]) + _dot(m, v1b[...]) + nb1[...], 0.0)
    hn = h + _dot(hid, v2[...]) + nb2[...]
    h_out[...] = hn
    ab_out[...] = jnp.concatenate(
        [_dot(hn, wa[...]) + ba[...], _dot(hn, wb[...])], axis=1)


def _node_last_body(h_ref, p0_ref, p1_ref, v1a, v1b, nb1, v2, nb2, win, bin_,
                    h_out, qkv_out):
    h = h_ref[...]
    m = p0_ref[...] + p1_ref[...]
    hid = jnp.maximum(_dot(h, v1a[...]) + _dot(m, v1b[...]) + nb1[...], 0.0)
    hn = h + _dot(hid, v2[...]) + nb2[...]
    h_out[...] = hn
    qkv_out[...] = _dot(hn, win[...]) + bin_[...]


def _node_update(h, p0, p1, v1a, v1b, nb1, v2, nb2, last, *proj):
    grid = (N // BT,)
    row = lambda i: (i, 0)
    const = lambda shape: pl.BlockSpec(shape, lambda i: (0,) * len(shape))
    tile = pl.BlockSpec((BT, H), row)
    if not last:
        wa, ba, wb = proj
        return pl.pallas_call(
            _node_body, grid=grid,
            in_specs=[tile, tile, tile, const((H, H)), const((H, H)),
                      const((1, H)), const((H, H)), const((1, H)),
                      const((H, H)), const((1, H)), const((H, H))],
            out_specs=[tile, pl.BlockSpec((BT, 2 * H), row)],
            out_shape=[jax.ShapeDtypeStruct((N, H), jnp.float32),
                       jax.ShapeDtypeStruct((N, 2 * H), jnp.float32)],
        )(h, p0, p1, v1a, v1b, nb1, v2, nb2, wa, ba, wb)
    win, bin_ = proj
    return pl.pallas_call(
        _node_last_body, grid=grid,
        in_specs=[tile, tile, tile, const((H, H)), const((H, H)),
                  const((1, H)), const((H, H)), const((1, H)),
                  const((H, 3 * H)), const((1, 3 * H))],
        out_specs=[tile, pl.BlockSpec((BT, 3 * H), row)],
        out_shape=[jax.ShapeDtypeStruct((N, H), jnp.float32),
                   jax.ShapeDtypeStruct((N, 3 * H), jnp.float32)],
    )(h, p0, p1, v1a, v1b, nb1, v2, nb2, win, bin_)


def _attn_body(lo_ref, hi_ref, qkv_ref, h_ref, tf_ref, bq_ref, br_ref,
               wout, bout, wf1, bf1, wf2, bf2, wd1, bd1, wd2, bd2, o_ref):
    t = pl.program_id(0)
    lo = lo_ref[t]
    hi = hi_ref[t]
    bq = bq_ref[...]
    scale = jnp.float32(1.0 / np.sqrt(DH))
    qt = qkv_ref[pl.ds(pl.multiple_of(t * TQ, TQ), TQ), :]
    qs = [qt[:, hd * DH:(hd + 1) * DH] * scale for hd in range(HEADS)]

    def body(j, carry):
        off = pl.multiple_of(j * TK, TK)
        blk = qkv_ref[pl.ds(off, TK), :]
        bk = br_ref[:, pl.ds(off, TK)]
        mask = bq == bk
        new = []
        for hd in range(HEADS):
            m, l_, acc = carry[hd]
            kh = blk[:, H + hd * DH:H + (hd + 1) * DH]
            vh = blk[:, 2 * H + hd * DH:2 * H + (hd + 1) * DH]
            s = lax.dot_general(qs[hd], kh, (((1,), (1,)), ((), ())),
                                preferred_element_type=jnp.float32)
            s = jnp.where(mask, s, jnp.float32(-1e9))
            mn = jnp.maximum(m, jnp.max(s, axis=1, keepdims=True))
            alpha = jnp.exp(m - mn)
            p = jnp.exp(s - mn)
            l2 = l_ * alpha + jnp.sum(p, axis=1, keepdims=True)
            acc2 = acc * alpha + _dot(p, vh)
            new.append((mn, l2, acc2))
        return tuple(new)

    init = tuple((jnp.full((TQ, 1), -1e30, jnp.float32),
                  jnp.zeros((TQ, 1), jnp.float32),
                  jnp.zeros((TQ, DH), jnp.float32)) for _ in range(HEADS))
    carry = lax.fori_loop(lo, hi, body, init)
    o = jnp.concatenate([acc / l_ for (m, l_, acc) in carry], axis=1)
    h = h_ref[...]
    ao = _dot(o, wout[...]) + bout[...]
    z = h + h + ao
    hf = _dot(jnp.maximum(_dot(z, wf1[...]) + bf1[...], 0.0), wf2[...]) + bf2[...]
    tval = tf_ref[...]
    res = jnp.zeros_like(o_ref)
    for i in range(3):
        cand = _dot(jnp.maximum(_dot(hf, wd1[i]) + bd1[i], 0.0), wd2[i]) + bd2[i]
        res = jnp.where(tval == jnp.float32(i), cand, res)
    o_ref[...] = res


def _attention(lo, hi, qkv, h, tf, bq, br, wout, bout, wf1, bf1, wf2, bf2,
               wd1, bd1, wd2, bd2):
    row = lambda i, *_: (i, 0)
    const = lambda shape: pl.BlockSpec(shape, lambda i, *_: (0,) * len(shape))
    grid_spec = pltpu.PrefetchScalarGridSpec(
        num_scalar_prefetch=2,
        grid=(NT,),
        in_specs=[
            const((NPAD, 3 * H)),
            pl.BlockSpec((TQ, H), row),
            pl.BlockSpec((TQ, 1), row),
            pl.BlockSpec((TQ, 1), row),
            const((1, NPAD)),
            const((H, H)), const((1, H)),
            const((H, H)), const((1, H)),
            const((H, H)), const((1, H)),
            const((3, H, H)), const((3, 1, H)),
            const((3, H, OUT_DIM)), const((3, 1, OUT_DIM)),
        ],
        out_specs=pl.BlockSpec((TQ, OUT_DIM), row),
    )
    return pl.pallas_call(
        _attn_body,
        grid_spec=grid_spec,
        out_shape=jax.ShapeDtypeStruct((NPAD, OUT_DIM), jnp.float32),
    )(lo, hi, qkv, h, tf, bq, br, wout, bout, wf1, bf1, wf2, bf2,
      wd1, bd1, wd2, bd2)


@functools.lru_cache(maxsize=1)
def _sc_gather_kernel():
    mesh = plsc.VectorSubcoreMesh(core_axis_name="c", subcore_axis_name="s")

    NB = 5

    @functools.partial(
        pl.kernel,
        mesh=mesh,
        out_type=[jax.ShapeDtypeStruct((EP, 2 * H), jnp.float32),
                  jax.ShapeDtypeStruct((EP, 2 * H), jnp.float32)],
        scratch_types=[
            pltpu.VMEM((NCH, CH), jnp.int32),
            pltpu.VMEM((NCH, CH), jnp.int32),
            pltpu.VMEM((NB, CH, 2 * H), jnp.float32),
            pltpu.VMEM((NB, CH, 2 * H), jnp.float32),
        ] + [pltpu.SemaphoreType.DMA] * (2 * NB),
    )
    def k(tab, isrc, idst, oa, ob, iv_s, iv_d, bufa, bufb, *sems):
        wid = lax.axis_index("s") * NC + lax.axis_index("c")
        base = wid * (NCH * CH)
        pltpu.sync_copy(isrc.at[wid], iv_s)
        pltpu.sync_copy(idst.at[wid], iv_d)
        sas = sems[:NB]
        sbs = sems[NB:]

        for b in range(NB):
            pltpu.async_copy(tab.at[iv_s.at[b]], bufa.at[b], sas[b])
            pltpu.async_copy(tab.at[iv_d.at[b]], bufb.at[b], sbs[b])

        def body(t, carry):
            for b in range(NB):
                j = t * NB + b
                pltpu.make_async_copy(tab.at[iv_s.at[j]], bufa.at[b],
                                      sas[b]).wait()
                pltpu.make_async_copy(tab.at[iv_d.at[j]], bufb.at[b],
                                      sbs[b]).wait()
                pltpu.sync_copy(bufa.at[b], oa.at[pl.ds(base + j * CH, CH)])
                pltpu.sync_copy(bufb.at[b], ob.at[pl.ds(base + j * CH, CH)])

                @pl.when(j + NB < NCH)
                def _():
                    pltpu.async_copy(tab.at[iv_s.at[j + NB]], bufa.at[b],
                                     sas[b])
                    pltpu.async_copy(tab.at[iv_d.at[j + NB]], bufb.at[b],
                                     sbs[b])
            return carry

        lax.fori_loop(0, NCH // NB, body, 0)

    return k


def _sc_gather2(tab, isrc, idst):
    return _sc_gather_kernel()(tab, isrc, idst)


@functools.lru_cache(maxsize=1)
def _sc_scatter_kernel():
    mesh = plsc.VectorSubcoreMesh(core_axis_name="c", subcore_axis_name="s")

    NB = 2

    @functools.partial(
        pl.kernel,
        mesh=mesh,
        out_type=jax.ShapeDtypeStruct((NC, ND, H), jnp.float32),
        scratch_types=[
            pltpu.VMEM((NCH, CH), jnp.int32),
            pltpu.VMEM((NB, CH, H), jnp.float32),
            pltpu.VMEM_SHARED((ND, H), jnp.float32),
            pltpu.SemaphoreType.DMA,
            pltpu.SemaphoreType.DMA,
        ],
    )
    def k(he, idst, zeros_nd, out, iv, buf, acc, s0, s1):
        cid = lax.axis_index("c")
        sid = lax.axis_index("s")
        wid = sid * NC + cid
        ebase = wid * NCH * CH
        pltpu.sync_copy(zeros_nd.at[pl.ds(sid * ROWS_PER_SUB, ROWS_PER_SUB)],
                        acc.at[pl.ds(sid * ROWS_PER_SUB, ROWS_PER_SUB)])
        pltpu.sync_copy(idst.at[wid], iv)
        plsc.subcore_barrier()
        sems = (s0, s1)

        for b in range(NB):
            pltpu.async_copy(he.at[pl.ds(ebase + b * CH, CH)], buf.at[b],
                             sems[b])

        def body(t, carry):
            for b in range(NB):
                j = t * NB + b
                pltpu.make_async_copy(he.at[pl.ds(ebase + j * CH, CH)],
                                      buf.at[b], sems[b]).wait()
                pltpu.sync_copy(buf.at[b], acc.at[iv.at[j]], add=True)

                @pl.when(j + NB < NCH)
                def _():
                    pltpu.async_copy(he.at[pl.ds(ebase + (j + NB) * CH, CH)],
                                     buf.at[b], sems[b])
            return carry

        lax.fori_loop(0, NCH // NB, body, 0)
        plsc.subcore_barrier()
        pltpu.sync_copy(acc.at[pl.ds(sid * ROWS_PER_SUB, ROWS_PER_SUB)],
                        out.at[cid, pl.ds(sid * ROWS_PER_SUB, ROWS_PER_SUB)])

    return k


def _sc_scatter(he, idst, zeros_nd):
    return _sc_scatter_kernel()(he, idst, zeros_nd)


def kernel(x, pe, params, edge_index, batch):
    f32 = jnp.float32
    inp = jnp.concatenate([x, pe], axis=-1)
    x2 = x[:, 2:3]
    src = edge_index[0]
    dst = edge_index[1]
    pad = EP - E
    src_p = jnp.concatenate([src, jnp.zeros((pad,), jnp.int32)])
    dst_p = jnp.concatenate([dst, jnp.zeros((pad,), jnp.int32)])
    dst_s = jnp.concatenate(
        [dst, N + (jnp.arange(pad, dtype=jnp.int32) % NS)])
    isrc = src_p.reshape(NW, NCH, CH)
    idst_g = dst_p.reshape(NW, NCH, CH)
    idst_s = dst_s.reshape(NW, NCH, CH)
    zeros_nd = jnp.zeros((ND, H), f32)

    p = params
    ew1 = jnp.stack([q["W1"].T for q in p["node_enc"]])
    eb1 = jnp.stack([q["b1"][None] for q in p["node_enc"]])
    ew2 = jnp.stack([q["W2"].T for q in p["node_enc"]])
    eb2 = jnp.stack([q["b2"][None] for q in p["node_enc"]])

    eu = p["edge_upd"]
    w1a = [q["W1"][:, :H].T for q in eu]
    w1b = [q["W1"][:, H:2 * H].T for q in eu]
    w1c = [q["W1"][:, 2 * H:].T for q in eu]
    ew2l = [q["W2"].T for q in eu]
    eb1l = [q["b1"][None] for q in eu]
    eb2l = [q["b2"][None] for q in eu]
    nu = p["node_upd"]
    v1a = [q["W1"][:, :H].T for q in nu]
    v1b = [q["W1"][:, H:].T for q in nu]
    nv2 = [q["W2"].T for q in nu]
    nb1 = [q["b1"][None] for q in nu]
    nb2 = [q["b2"][None] for q in nu]
    ebias = p["edge_bias"][None]

    h, ab_tab, tf = _encode(inp, x2, ew1, eb1, ew2, eb2,
                            w1a[0], eb1l[0], w1b[0])

    he = None
    qkv = None
    for l in range(NUM_LAYERS):
        gsrc, gdst = _sc_gather2(ab_tab, isrc, idst_g)
        he = _edge_update(gsrc, gdst, he, w1c[l], ew2l[l], eb2l[l], ebias)
        parts = _sc_scatter(he, idst_s, zeros_nd)
        p0, p1 = parts[0], parts[1]
        if l < NUM_LAYERS - 1:
            h, ab_tab = _node_update(
                h, p0, p1, v1a[l], v1b[l], nb1[l], nv2[l], nb2[l], False,
                w1a[l + 1], eb1l[l + 1], w1b[l + 1])
        else:
            h, qkv = _node_update(
                h, p0, p1, v1a[l], v1b[l], nb1[l], nv2[l], nb2[l], True,
                p["attn_in_W"].T, p["attn_in_b"][None])

    npad = NPAD - N
    batch_p = jnp.concatenate(
        [batch.astype(jnp.int32), jnp.full((npad,), 99, jnp.int32)])
    gid = jnp.arange(G, dtype=batch.dtype)
    starts = jnp.searchsorted(batch, gid, side="left").astype(jnp.int32)
    ends = jnp.searchsorted(batch, gid, side="right").astype(jnp.int32)
    t0 = jnp.arange(NT, dtype=jnp.int32) * TQ
    gmin = jnp.clip(batch_p[t0], 0, G - 1)
    gmax = jnp.clip(batch_p[t0 + TQ - 1], 0, G - 1)
    lo = starts[gmin] // TK
    hi = (ends[gmax] + TK - 1) // TK
    bq = batch_p.astype(f32)[:, None]
    br = batch_p.astype(f32)[None, :]
    qkv_p = jnp.pad(qkv, ((0, npad), (0, 0)))
    h_p = jnp.pad(h, ((0, npad), (0, 0)))
    tf_p = jnp.pad(tf, ((0, npad), (0, 0)), constant_values=99.0)

    out = _attention(
        lo, hi, qkv_p, h_p, tf_p, bq, br,
        p["attn_out_W"].T, p["attn_out_b"][None],
        p["fusion"]["W1"].T, p["fusion"]["b1"][None],
        p["fusion"]["W2"].T, p["fusion"]["b2"][None],
        jnp.stack([q["W1"].T for q in p["dec"]]),
        jnp.stack([q["b1"][None] for q in p["dec"]]),
        jnp.stack([q["W2"].T for q in p["dec"]]),
        jnp.stack([q["b2"][None] for q in p["dec"]]),
    )
    return out[:N]
